# Initial kernel scaffold; baseline (speedup 1.0000x reference)
import jax
import jax.numpy as jnp
from jax.experimental import pallas as pl


def kernel(xs, ys, cands, table):
    raise NotImplementedError("write your pallas kernel here")


# speedup vs baseline: 2.4673x; 2.4673x over previous
import functools

import jax
import jax.numpy as jnp
from jax import lax
from jax.experimental import pallas as pl
from jax.experimental.pallas import tpu as pltpu
from jax.experimental.pallas import tpu_sc as plsc

VOCAB = 1000000
DIM = 64
B = 1024
L = 50
NC = 20

NSETS = NC + 2
NENC = NSETS * B
NWORKERS = 32
ENC_PER_W = NENC // NWORKERS
CHUNK = 16
NSTEPS = ENC_PER_W // CHUNK
GW = 80
NGATHER = (CHUNK * L) // GW
LANES = 16
DSUB = DIM // LANES


def _sc_encode_sums(idx4, table):
    mesh = plsc.VectorSubcoreMesh(core_axis_name="c", subcore_axis_name="s")

    @functools.partial(
        pl.kernel,
        out_type=jax.ShapeDtypeStruct((NENC, DIM), jnp.float32),
        mesh=mesh,
        scratch_types=[
            pltpu.VMEM((NGATHER, GW), jnp.int32),
            pltpu.VMEM((CHUNK * L, DIM), jnp.float32),
            pltpu.VMEM((CHUNK, DIM), jnp.float32),
            pltpu.SemaphoreType.DMA,
        ],
        compiler_params=pltpu.CompilerParams(use_tc_tiling_on_sc=False),
    )
    def sc_kernel(idx_hbm, table_hbm, out_hbm, idx_v, rows_v, out_v, sem):
        wid = lax.axis_index("s") * 2 + lax.axis_index("c")

        @pl.loop(0, NSTEPS)
        def _step(s):
            pltpu.sync_copy(idx_hbm.at[wid, s], idx_v)
            cps = [
                pltpu.async_copy(
                    table_hbm.at[idx_v.at[c]],
                    rows_v.at[pl.ds(c * GW, GW)],
                    sem,
                )
                for c in range(NGATHER)
            ]
            for cp in cps:
                cp.wait()

            @pl.loop(0, CHUNK)
            def _enc(e):
                base = e * L
                for c4 in range(DSUB):
                    acc = rows_v[base, pl.ds(c4 * LANES, LANES)]
                    for l in range(1, L):
                        acc = acc + rows_v[base + l, pl.ds(c4 * LANES, LANES)]
                    out_v[e, pl.ds(c4 * LANES, LANES)] = acc

            pltpu.sync_copy(
                out_v, out_hbm.at[pl.ds((wid * NSTEPS + s) * CHUNK, CHUNK)]
            )

    return sc_kernel(idx4, table)


def _tc_score_softmax(xs_sums, enc_sums):
    inv = 1.0 / float(L * L)

    def body(x_ref, e_ref, o_ref):
        x = x_ref[...] * inv
        e = e_ref[0]
        s = lax.dot_general(
            x, e, (((1,), (1,)), ((), ())),
            preferred_element_type=jnp.float32,
            precision=lax.Precision.HIGHEST,
        )
        m = jnp.max(s, axis=1, keepdims=True)
        p = jnp.exp(s - m)
        o_ref[...] = p / jnp.sum(p, axis=1, keepdims=True)

    nblk = NSETS - 1
    return pl.pallas_call(
        body,
        grid=(nblk,),
        in_specs=[
            pl.BlockSpec((B, DIM), lambda k: (0, 0)),
            pl.BlockSpec((1, B, DIM), lambda k: (k, 0, 0)),
        ],
        out_specs=pl.BlockSpec((B, B), lambda k: (k, 0)),
        out_shape=jax.ShapeDtypeStruct((nblk * B, B), jnp.float32),
    )(xs_sums, enc_sums)


def kernel(xs, ys, cands, table):
    idx = jnp.concatenate(
        [xs.reshape(-1), ys.reshape(-1), cands.reshape(-1)]
    ).astype(jnp.int32)
    idx4 = idx.reshape(NWORKERS, NSTEPS, NGATHER, GW)
    sums = _sc_encode_sums(idx4, table)
    xs_sums = sums[:B]
    enc_sums = sums[B:].reshape(NSETS - 1, B, DIM)
    pred = _tc_score_softmax(xs_sums, enc_sums)
    return pred[None]


# speedup vs baseline: 2.9202x; 1.1836x over previous
import functools

import jax
import jax.numpy as jnp
from jax import lax
from jax.experimental import pallas as pl
from jax.experimental.pallas import tpu as pltpu
from jax.experimental.pallas import tpu_sc as plsc

VOCAB = 1000000
DIM = 64
B = 1024
L = 50
NC = 20

NSETS = NC + 2
NENC = NSETS * B
NWORKERS = 32
ENC_PER_W = NENC // NWORKERS
CHUNK = 16
NSTEPS = ENC_PER_W // CHUNK
ROWS = CHUNK * L
GW = 80
NGATHER = ROWS // GW
LANES = 16
DSUB = DIM // LANES


def _sc_encode_sums(idx, table):
    mesh = plsc.VectorSubcoreMesh(core_axis_name="c", subcore_axis_name="s")

    @functools.partial(
        pl.kernel,
        out_type=jax.ShapeDtypeStruct((NENC, DIM), jnp.float32),
        mesh=mesh,
        scratch_types=[
            pltpu.VMEM((2, ROWS), jnp.int32),
            pltpu.VMEM((2, ROWS, DIM), jnp.float32),
            pltpu.VMEM((CHUNK, DIM), jnp.float32),
            pltpu.SemaphoreType.DMA,
            pltpu.SemaphoreType.DMA,
        ],
        compiler_params=pltpu.CompilerParams(use_tc_tiling_on_sc=False),
    )
    def sc_kernel(idx_hbm, table_hbm, out_hbm, idx_v, rows_v, out_v, sem0, sem1):
        wid = lax.axis_index("s") * 2 + lax.axis_index("c")
        sems = (sem0, sem1)

        def stage_and_fire(s, b):
            pltpu.sync_copy(idx_hbm.at[pl.ds((wid * NSTEPS + s) * ROWS, ROWS)],
                            idx_v.at[b])
            for c in range(NGATHER):
                pltpu.async_copy(
                    table_hbm.at[idx_v.at[b, pl.ds(c * GW, GW)]],
                    rows_v.at[b, pl.ds(c * GW, GW)],
                    sems[b],
                )

        def drain(b):
            for c in range(NGATHER):
                pltpu.make_async_copy(
                    table_hbm.at[idx_v.at[b, pl.ds(c * GW, GW)]],
                    rows_v.at[b, pl.ds(c * GW, GW)],
                    sems[b],
                ).wait()

        def reduce_store(s, b):
            @pl.loop(0, CHUNK)
            def _enc(e):
                base = e * L
                for c4 in range(DSUB):
                    acc = rows_v[b, base, pl.ds(c4 * LANES, LANES)]
                    for l in range(1, L):
                        acc = acc + rows_v[b, base + l, pl.ds(c4 * LANES, LANES)]
                    out_v[e, pl.ds(c4 * LANES, LANES)] = acc

            pltpu.sync_copy(
                out_v, out_hbm.at[pl.ds((wid * NSTEPS + s) * CHUNK, CHUNK)]
            )

        stage_and_fire(0, 0)

        @pl.loop(0, NSTEPS, step=2)
        def _step(s):
            stage_and_fire(s + 1, 1)
            drain(0)
            reduce_store(s, 0)

            @pl.when(s + 2 < NSTEPS)
            def _():
                stage_and_fire(s + 2, 0)

            drain(1)
            reduce_store(s + 1, 1)

    return sc_kernel(idx, table)


def _tc_score_softmax(xs_sums, enc_sums):
    inv = 1.0 / float(L * L)

    def body(x_ref, e_ref, o_ref):
        x = x_ref[...] * inv
        e = e_ref[0]
        s = lax.dot_general(
            x, e, (((1,), (1,)), ((), ())),
            preferred_element_type=jnp.float32,
            precision=lax.Precision.HIGHEST,
        )
        m = jnp.max(s, axis=1, keepdims=True)
        p = jnp.exp(s - m)
        o_ref[...] = p / jnp.sum(p, axis=1, keepdims=True)

    nblk = NSETS - 1
    return pl.pallas_call(
        body,
        grid=(nblk,),
        in_specs=[
            pl.BlockSpec((B, DIM), lambda k: (0, 0)),
            pl.BlockSpec((1, B, DIM), lambda k: (k, 0, 0)),
        ],
        out_specs=pl.BlockSpec((B, B), lambda k: (k, 0)),
        out_shape=jax.ShapeDtypeStruct((nblk * B, B), jnp.float32),
    )(xs_sums, enc_sums)


def kernel(xs, ys, cands, table):
    idx = jnp.concatenate(
        [xs.reshape(-1), ys.reshape(-1), cands.reshape(-1)]
    ).astype(jnp.int32)
    sums = _sc_encode_sums(idx, table)
    xs_sums = sums[:B]
    enc_sums = sums[B:].reshape(NSETS - 1, B, DIM)
    pred = _tc_score_softmax(xs_sums, enc_sums)
    return pred[None]


# speedup vs baseline: 3.7140x; 1.2718x over previous
import functools

import jax
import jax.numpy as jnp
from jax import lax
from jax.experimental import pallas as pl
from jax.experimental.pallas import tpu as pltpu
from jax.experimental.pallas import tpu_sc as plsc

VOCAB = 1000000
DIM = 64
B = 1024
L = 50
NC = 20

NSETS = NC + 2
NENC = NSETS * B
NWORKERS = 32
ENC_PER_W = NENC // NWORKERS
CHUNK = 16
NSTEPS = ENC_PER_W // CHUNK
ROWS = CHUNK * L
GW = 80
NGATHER = ROWS // GW
LANES = 16
DSUB = DIM // LANES


def _sc_encode_sums(idx, table):
    mesh = plsc.VectorSubcoreMesh(core_axis_name="c", subcore_axis_name="s")

    @functools.partial(
        pl.kernel,
        out_type=jax.ShapeDtypeStruct((NENC, DIM), jnp.float32),
        mesh=mesh,
        scratch_types=[
            pltpu.VMEM((2, ROWS), jnp.int32),
            pltpu.VMEM((2, ROWS, DIM), jnp.float32),
            pltpu.VMEM((CHUNK, DIM), jnp.float32),
            pltpu.SemaphoreType.DMA,
            pltpu.SemaphoreType.DMA,
        ],
        compiler_params=pltpu.CompilerParams(use_tc_tiling_on_sc=False),
    )
    def sc_kernel(idx_hbm, table_hbm, out_hbm, idx_v, rows_v, out_v, sem0, sem1):
        wid = lax.axis_index("s") * 2 + lax.axis_index("c")
        sems = (sem0, sem1)

        def stage_and_fire(s, b):
            pltpu.sync_copy(idx_hbm.at[pl.ds((wid * NSTEPS + s) * ROWS, ROWS)],
                            idx_v.at[b])
            for c in range(NGATHER):
                pltpu.async_copy(
                    table_hbm.at[idx_v.at[b, pl.ds(c * GW, GW)]],
                    rows_v.at[b, pl.ds(c * GW, GW)],
                    sems[b],
                )

        def drain(b):
            for c in range(NGATHER):
                pltpu.make_async_copy(
                    table_hbm.at[idx_v.at[b, pl.ds(c * GW, GW)]],
                    rows_v.at[b, pl.ds(c * GW, GW)],
                    sems[b],
                ).wait()

        def reduce_store(s, b):
            @pl.loop(0, CHUNK)
            def _enc(e):
                base = e * L
                for c4 in range(DSUB):
                    acc = rows_v[b, base, pl.ds(c4 * LANES, LANES)]
                    for l in range(1, L):
                        acc = acc + rows_v[b, base + l, pl.ds(c4 * LANES, LANES)]
                    out_v[e, pl.ds(c4 * LANES, LANES)] = acc

            pltpu.sync_copy(
                out_v, out_hbm.at[pl.ds((wid * NSTEPS + s) * CHUNK, CHUNK)]
            )

        stage_and_fire(0, 0)

        @pl.loop(0, NSTEPS, step=2)
        def _step(s):
            stage_and_fire(s + 1, 1)
            drain(0)
            reduce_store(s, 0)

            @pl.when(s + 2 < NSTEPS)
            def _():
                stage_and_fire(s + 2, 0)

            drain(1)
            reduce_store(s + 1, 1)

    return sc_kernel(idx, table)


RELAY_CB = 4096
RELAY_GRID = -(-VOCAB // RELAY_CB)


def _tc_relayout(tt):
    def body(t_ref, o_ref, scr):
        scr[...] = jnp.transpose(t_ref[...])
        a = scr[pl.Slice(0, RELAY_CB // 2, 2), :]
        b = scr[pl.Slice(1, RELAY_CB // 2, 2), :]
        o_ref[...] = jnp.concatenate([a, b], axis=1)

    return pl.pallas_call(
        body,
        grid=(RELAY_GRID,),
        in_specs=[pl.BlockSpec((DIM, RELAY_CB), lambda i: (0, i))],
        out_specs=pl.BlockSpec((RELAY_CB // 2, 2 * DIM), lambda i: (i, 0)),
        out_shape=jax.ShapeDtypeStruct((VOCAB // 2, 2 * DIM), jnp.float32),
        scratch_shapes=[pltpu.VMEM((RELAY_CB, DIM), jnp.float32)],
    )(tt)


def _tc_score_softmax(xs_sums, enc_sums):
    inv = 1.0 / float(L * L)

    def body(x_ref, e_ref, o_ref):
        x = x_ref[...] * inv
        e = e_ref[0]
        s = lax.dot_general(
            x, e, (((1,), (1,)), ((), ())),
            preferred_element_type=jnp.float32,
            precision=lax.Precision.HIGHEST,
        )
        m = jnp.max(s, axis=1, keepdims=True)
        p = jnp.exp(s - m)
        o_ref[...] = p / jnp.sum(p, axis=1, keepdims=True)

    nblk = NSETS - 1
    return pl.pallas_call(
        body,
        grid=(nblk,),
        in_specs=[
            pl.BlockSpec((B, DIM), lambda k: (0, 0)),
            pl.BlockSpec((1, B, DIM), lambda k: (k, 0, 0)),
        ],
        out_specs=pl.BlockSpec((B, B), lambda k: (k, 0)),
        out_shape=jax.ShapeDtypeStruct((nblk * B, B), jnp.float32),
    )(xs_sums, enc_sums)


def kernel(xs, ys, cands, table):
    idx = jnp.concatenate(
        [xs.reshape(-1), ys.reshape(-1), cands.reshape(-1)]
    ).astype(jnp.int32)
    t_lin = _tc_relayout(table.T).reshape(VOCAB, DIM)
    sums = _sc_encode_sums(idx, t_lin)
    xs_sums = sums[:B]
    enc_sums = sums[B:].reshape(NSETS - 1, B, DIM)
    pred = _tc_score_softmax(xs_sums, enc_sums)
    return pred[None]


# speedup vs baseline: 3.9081x; 1.0522x over previous
import functools

import jax
import jax.numpy as jnp
from jax import lax
from jax.experimental import pallas as pl
from jax.experimental.pallas import tpu as pltpu
from jax.experimental.pallas import tpu_sc as plsc

VOCAB = 1000000
DIM = 64
B = 1024
L = 50
NC = 20

NSETS = NC + 2
NENC = NSETS * B
NWORKERS = 32
ENC_PER_W = NENC // NWORKERS
CHUNK = 16
NSTEPS = ENC_PER_W // CHUNK
ROWS = CHUNK * L
GW = 80
NGATHER = ROWS // GW
LANES = 16
DSUB = DIM // LANES


def _sc_encode_sums(idx, table):
    mesh = plsc.VectorSubcoreMesh(core_axis_name="c", subcore_axis_name="s")

    @functools.partial(
        pl.kernel,
        out_type=jax.ShapeDtypeStruct((NENC, DIM), jnp.float32),
        mesh=mesh,
        scratch_types=[
            pltpu.VMEM((2, ROWS), jnp.int32),
            pltpu.VMEM((2, ROWS, DIM), jnp.float32),
            pltpu.VMEM((CHUNK, DIM), jnp.float32),
            pltpu.SemaphoreType.DMA,
            pltpu.SemaphoreType.DMA,
        ],
        compiler_params=pltpu.CompilerParams(use_tc_tiling_on_sc=False),
    )
    def sc_kernel(idx_hbm, table_hbm, out_hbm, idx_v, rows_v, out_v, sem0, sem1):
        wid = lax.axis_index("s") * 2 + lax.axis_index("c")
        sems = (sem0, sem1)

        def stage_and_fire(s, b):
            pltpu.sync_copy(idx_hbm.at[pl.ds((wid * NSTEPS + s) * ROWS, ROWS)],
                            idx_v.at[b])
            for c in range(NGATHER):
                pltpu.async_copy(
                    table_hbm.at[idx_v.at[b, pl.ds(c * GW, GW)]],
                    rows_v.at[b, pl.ds(c * GW, GW)],
                    sems[b],
                )

        def drain(b):
            for c in range(NGATHER):
                pltpu.make_async_copy(
                    table_hbm.at[idx_v.at[b, pl.ds(c * GW, GW)]],
                    rows_v.at[b, pl.ds(c * GW, GW)],
                    sems[b],
                ).wait()

        def reduce_store(s, b):
            @pl.loop(0, CHUNK)
            def _enc(e):
                base = e * L
                for c4 in range(DSUB):
                    acc = rows_v[b, base, pl.ds(c4 * LANES, LANES)]
                    for l in range(1, L):
                        acc = acc + rows_v[b, base + l, pl.ds(c4 * LANES, LANES)]
                    out_v[e, pl.ds(c4 * LANES, LANES)] = acc

            pltpu.sync_copy(
                out_v, out_hbm.at[pl.ds((wid * NSTEPS + s) * CHUNK, CHUNK)]
            )

        stage_and_fire(0, 0)

        @pl.loop(0, NSTEPS, step=2)
        def _step(s):
            stage_and_fire(s + 1, 1)
            drain(0)
            reduce_store(s, 0)

            @pl.when(s + 2 < NSTEPS)
            def _():
                stage_and_fire(s + 2, 0)

            drain(1)
            reduce_store(s + 1, 1)

    return sc_kernel(idx, table)


RELAY_CB = 8192
RELAY_GRID = -(-VOCAB // RELAY_CB)


def _tc_relayout(tt):
    def body(t_ref, o_ref, scr):
        scr[...] = jnp.transpose(t_ref[...])
        o_ref[:, 0:DIM] = scr[pl.Slice(0, RELAY_CB // 2, 2), :]
        o_ref[:, DIM:2 * DIM] = scr[pl.Slice(1, RELAY_CB // 2, 2), :]

    return pl.pallas_call(
        body,
        grid=(RELAY_GRID,),
        in_specs=[pl.BlockSpec((DIM, RELAY_CB), lambda i: (0, i))],
        out_specs=pl.BlockSpec((RELAY_CB // 2, 2 * DIM), lambda i: (i, 0)),
        out_shape=jax.ShapeDtypeStruct((VOCAB // 2, 2 * DIM), jnp.float32),
        scratch_shapes=[pltpu.VMEM((RELAY_CB, DIM), jnp.float32)],
    )(tt)


def _tc_score_softmax(xs_sums, enc_sums):
    inv = 1.0 / float(L * L)

    def body(x_ref, e_ref, o_ref):
        x = x_ref[...] * inv
        e = e_ref[0]
        s = lax.dot_general(
            x, e, (((1,), (1,)), ((), ())),
            preferred_element_type=jnp.float32,
            precision=lax.Precision.HIGHEST,
        )
        m = jnp.max(s, axis=1, keepdims=True)
        p = jnp.exp(s - m)
        o_ref[...] = p / jnp.sum(p, axis=1, keepdims=True)

    nblk = NSETS - 1
    return pl.pallas_call(
        body,
        grid=(nblk,),
        in_specs=[
            pl.BlockSpec((B, DIM), lambda k: (0, 0)),
            pl.BlockSpec((1, B, DIM), lambda k: (k, 0, 0)),
        ],
        out_specs=pl.BlockSpec((B, B), lambda k: (k, 0)),
        out_shape=jax.ShapeDtypeStruct((nblk * B, B), jnp.float32),
    )(xs_sums, enc_sums)


def kernel(xs, ys, cands, table):
    idx = jnp.concatenate(
        [xs.reshape(-1), ys.reshape(-1), cands.reshape(-1)]
    ).astype(jnp.int32)
    t_lin = _tc_relayout(table.T).reshape(VOCAB, DIM)
    sums = _sc_encode_sums(idx, t_lin)
    xs_sums = sums[:B]
    enc_sums = sums[B:].reshape(NSETS - 1, B, DIM)
    pred = _tc_score_softmax(xs_sums, enc_sums)
    return pred[None]


# speedup vs baseline: 4.5024x; 1.1521x over previous
import functools

import jax
import jax.numpy as jnp
from jax import lax
from jax.experimental import pallas as pl
from jax.experimental.pallas import tpu as pltpu
from jax.experimental.pallas import tpu_sc as plsc

VOCAB = 1000000
DIM = 64
B = 1024
L = 50
NC = 20

NSETS = NC + 2
NENC = NSETS * B
NWORKERS = 32
ENC_PER_W = NENC // NWORKERS
CHUNK = 16
NSTEPS = ENC_PER_W // CHUNK
ROWS = CHUNK * L
GW = 80
NGATHER = ROWS // GW
LANES = 16
DSUB = DIM // LANES


def _sc_encode_sums(idx, table):
    mesh = plsc.VectorSubcoreMesh(core_axis_name="c", subcore_axis_name="s")

    @functools.partial(
        pl.kernel,
        out_type=jax.ShapeDtypeStruct((NENC, DIM), jnp.float32),
        mesh=mesh,
        scratch_types=[
            pltpu.VMEM((2, ROWS), jnp.int32),
            pltpu.VMEM((2, ROWS, DIM), jnp.float32),
            pltpu.VMEM((2, CHUNK, DIM), jnp.float32),
            pltpu.SemaphoreType.DMA,
            pltpu.SemaphoreType.DMA,
            pltpu.SemaphoreType.DMA,
            pltpu.SemaphoreType.DMA,
            pltpu.SemaphoreType.DMA,
            pltpu.SemaphoreType.DMA,
        ],
        compiler_params=pltpu.CompilerParams(use_tc_tiling_on_sc=False),
    )
    def sc_kernel(idx_hbm, table_hbm, out_hbm, idx_v, rows_v, out_v,
                  gsem0, gsem1, isem0, isem1, osem0, osem1):
        wid = lax.axis_index("s") * 2 + lax.axis_index("c")
        gsems = (gsem0, gsem1)
        isems = (isem0, isem1)
        osems = (osem0, osem1)

        def idx_copy(s, b):
            return pltpu.make_async_copy(
                idx_hbm.at[pl.ds((wid * NSTEPS + s) * ROWS, ROWS)],
                idx_v.at[b], isems[b])

        def out_copy(s, b):
            return pltpu.make_async_copy(
                out_v.at[b],
                out_hbm.at[pl.ds((wid * NSTEPS + s) * CHUNK, CHUNK)],
                osems[b])

        def gather_copy(c, b):
            return pltpu.make_async_copy(
                table_hbm.at[idx_v.at[b, pl.ds(c * GW, GW)]],
                rows_v.at[b, pl.ds(c * GW, GW)], gsems[b])

        def fire(b):
            for c in range(NGATHER):
                gather_copy(c, b).start()

        def drain(b):
            for c in range(NGATHER):
                gather_copy(c, b).wait()

        def reduce(s, b):
            @pl.when(s >= 2)
            def _():
                out_copy(s, b).wait()

            @pl.loop(0, CHUNK)
            def _enc(e):
                base = e * L
                for c4 in range(DSUB):
                    acc = rows_v[b, base, pl.ds(c4 * LANES, LANES)]
                    for l in range(1, L):
                        acc = acc + rows_v[b, base + l, pl.ds(c4 * LANES, LANES)]
                    out_v[b, e, pl.ds(c4 * LANES, LANES)] = acc

            out_copy(s, b).start()

        idx_copy(0, 0).start()
        idx_copy(1, 1).start()
        idx_copy(0, 0).wait()
        fire(0)

        @pl.loop(0, NSTEPS, step=2)
        def _step(s):
            idx_copy(s + 1, 1).wait()
            fire(1)
            drain(0)

            @pl.when(s + 2 < NSTEPS)
            def _():
                idx_copy(s + 2, 0).start()

            reduce(s, 0)

            @pl.when(s + 2 < NSTEPS)
            def _():
                idx_copy(s + 2, 0).wait()
                fire(0)

            drain(1)

            @pl.when(s + 3 < NSTEPS)
            def _():
                idx_copy(s + 3, 1).start()

            reduce(s + 1, 1)

        out_copy(NSTEPS - 2, 0).wait()
        out_copy(NSTEPS - 1, 1).wait()

    return sc_kernel(idx, table)


RELAY_CB = 8192
RELAY_GRID = -(-VOCAB // RELAY_CB)


def _tc_relayout(tt):
    def body(t_ref, o_ref, scr):
        scr[...] = jnp.transpose(t_ref[...])
        o_ref[:, 0:DIM] = scr[pl.Slice(0, RELAY_CB // 2, 2), :]
        o_ref[:, DIM:2 * DIM] = scr[pl.Slice(1, RELAY_CB // 2, 2), :]

    return pl.pallas_call(
        body,
        grid=(RELAY_GRID,),
        in_specs=[pl.BlockSpec((DIM, RELAY_CB), lambda i: (0, i))],
        out_specs=pl.BlockSpec((RELAY_CB // 2, 2 * DIM), lambda i: (i, 0)),
        out_shape=jax.ShapeDtypeStruct((VOCAB // 2, 2 * DIM), jnp.float32),
        scratch_shapes=[pltpu.VMEM((RELAY_CB, DIM), jnp.float32)],
    )(tt)


def _tc_score_softmax(sums):
    inv = 1.0 / float(L * L)

    def body(x_ref, e_ref, o_ref):
        x = x_ref[...] * inv
        e = e_ref[...]
        s = lax.dot_general(
            x, e, (((1,), (1,)), ((), ())),
            preferred_element_type=jnp.float32,
        )
        m = jnp.max(s, axis=1, keepdims=True)
        p = jnp.exp(s - m)
        o_ref[...] = p / jnp.sum(p, axis=1, keepdims=True)

    nblk = NSETS - 1
    return pl.pallas_call(
        body,
        grid=(nblk,),
        in_specs=[
            pl.BlockSpec((B, DIM), lambda k: (0, 0)),
            pl.BlockSpec((B, DIM), lambda k: (k + 1, 0)),
        ],
        out_specs=pl.BlockSpec((B, B), lambda k: (k, 0)),
        out_shape=jax.ShapeDtypeStruct((nblk * B, B), jnp.float32),
    )(sums, sums)


def kernel(xs, ys, cands, table):
    idx = jnp.concatenate(
        [xs.reshape(-1), ys.reshape(-1), cands.reshape(-1)]
    ).astype(jnp.int32)
    t_lin = _tc_relayout(table.T).reshape(VOCAB, DIM)
    sums = _sc_encode_sums(idx, t_lin)
    pred = _tc_score_softmax(sums)
    return pred[None]


# speedup vs baseline: 4.7430x; 1.0534x over previous
import functools

import jax
import jax.numpy as jnp
from jax import lax
from jax.experimental import pallas as pl
from jax.experimental.pallas import tpu as pltpu
from jax.experimental.pallas import tpu_sc as plsc

VOCAB = 1000000
DIM = 64
B = 1024
L = 50
NC = 20

NSETS = NC + 2
NENC = NSETS * B
NWORKERS = 32
ENC_PER_W = NENC // NWORKERS
CHUNK = 16
NSTEPS = ENC_PER_W // CHUNK
ROWS = CHUNK * L
GW = 80
NGATHER = ROWS // GW
LANES = 16
DSUB = DIM // LANES


def _sc_encode_sums(idx, table):
    mesh = plsc.VectorSubcoreMesh(core_axis_name="c", subcore_axis_name="s")

    @functools.partial(
        pl.kernel,
        out_type=jax.ShapeDtypeStruct((NENC, DIM), jnp.float32),
        mesh=mesh,
        scratch_types=[
            pltpu.VMEM((2, ROWS), jnp.int32),
            pltpu.VMEM((2, ROWS, DIM), jnp.float32),
            pltpu.VMEM((2, CHUNK, DIM), jnp.float32),
            pltpu.SemaphoreType.DMA,
            pltpu.SemaphoreType.DMA,
            pltpu.SemaphoreType.DMA,
            pltpu.SemaphoreType.DMA,
            pltpu.SemaphoreType.DMA,
            pltpu.SemaphoreType.DMA,
        ],
        compiler_params=pltpu.CompilerParams(use_tc_tiling_on_sc=False),
    )
    def sc_kernel(idx_hbm, table_hbm, out_hbm, idx_v, rows_v, out_v,
                  gsem0, gsem1, isem0, isem1, osem0, osem1):
        wid = lax.axis_index("s") * 2 + lax.axis_index("c")
        gsems = (gsem0, gsem1)
        isems = (isem0, isem1)
        osems = (osem0, osem1)

        def idx_copy(s, b):
            return pltpu.make_async_copy(
                idx_hbm.at[pl.ds((wid * NSTEPS + s) * ROWS, ROWS)],
                idx_v.at[b], isems[b])

        def out_copy(s, b):
            return pltpu.make_async_copy(
                out_v.at[b],
                out_hbm.at[pl.ds((wid * NSTEPS + s) * CHUNK, CHUNK)],
                osems[b])

        def gather_copy(c, b):
            return pltpu.make_async_copy(
                table_hbm.at[idx_v.at[b, pl.ds(c * GW, GW)]],
                rows_v.at[b, pl.ds(c * GW, GW)], gsems[b])

        def fire(b):
            for c in range(NGATHER):
                gather_copy(c, b).start()

        def drain(b):
            for c in range(NGATHER):
                gather_copy(c, b).wait()

        def reduce(s, b):
            @pl.when(s >= 2)
            def _():
                out_copy(s, b).wait()

            @pl.loop(0, CHUNK)
            def _enc(e):
                base = e * L
                for c4 in range(DSUB):
                    acc = rows_v[b, base, pl.ds(c4 * LANES, LANES)]
                    for l in range(1, L):
                        acc = acc + rows_v[b, base + l, pl.ds(c4 * LANES, LANES)]
                    out_v[b, e, pl.ds(c4 * LANES, LANES)] = acc

            out_copy(s, b).start()

        idx_copy(0, 0).start()
        idx_copy(1, 1).start()
        idx_copy(0, 0).wait()
        fire(0)

        @pl.loop(0, NSTEPS, step=2)
        def _step(s):
            idx_copy(s + 1, 1).wait()
            fire(1)
            drain(0)

            @pl.when(s + 2 < NSTEPS)
            def _():
                idx_copy(s + 2, 0).start()

            reduce(s, 0)

            @pl.when(s + 2 < NSTEPS)
            def _():
                idx_copy(s + 2, 0).wait()
                fire(0)

            drain(1)

            @pl.when(s + 3 < NSTEPS)
            def _():
                idx_copy(s + 3, 1).start()

            reduce(s + 1, 1)

        out_copy(NSTEPS - 2, 0).wait()
        out_copy(NSTEPS - 1, 1).wait()

    return sc_kernel(idx, table)


RELAY_CB = 8192
RELAY_GRID = -(-VOCAB // RELAY_CB)
V_PAD = RELAY_GRID * RELAY_CB
HB = RELAY_CB // 2


def _tc_relayout(tt):
    def body(t_ref, o_ref):
        y = jnp.transpose(t_ref[...])
        o_ref[:, 0:DIM] = y[0:HB]
        o_ref[:, DIM:2 * DIM] = y[HB:2 * HB]

    return pl.pallas_call(
        body,
        grid=(RELAY_GRID,),
        in_specs=[pl.BlockSpec((DIM, RELAY_CB), lambda i: (0, i))],
        out_specs=pl.BlockSpec((HB, 2 * DIM), lambda i: (i, 0)),
        out_shape=jax.ShapeDtypeStruct((V_PAD // 2, 2 * DIM), jnp.float32),
    )(tt)


def _remap_idx(r):
    return ((r >> 13) << 13) | ((r & 4095) << 1) | ((r >> 12) & 1)


def _tc_score_softmax(sums):
    inv = 1.0 / float(L * L)

    def body(x_ref, e_ref, o_ref):
        x = x_ref[...] * inv
        e = e_ref[...]
        s = lax.dot_general(
            x, e, (((1,), (1,)), ((), ())),
            preferred_element_type=jnp.float32,
        )
        m = jnp.max(s, axis=1, keepdims=True)
        p = jnp.exp(s - m)
        o_ref[...] = p / jnp.sum(p, axis=1, keepdims=True)

    nblk = NSETS - 1
    return pl.pallas_call(
        body,
        grid=(nblk,),
        in_specs=[
            pl.BlockSpec((B, DIM), lambda k: (0, 0)),
            pl.BlockSpec((B, DIM), lambda k: (k + 1, 0)),
        ],
        out_specs=pl.BlockSpec((B, B), lambda k: (k, 0)),
        out_shape=jax.ShapeDtypeStruct((nblk * B, B), jnp.float32),
    )(sums, sums)


def kernel(xs, ys, cands, table):
    idx = jnp.concatenate(
        [xs.reshape(-1), ys.reshape(-1), cands.reshape(-1)]
    ).astype(jnp.int32)
    idx = _remap_idx(idx)
    t_lin = _tc_relayout(table.T).reshape(V_PAD, DIM)
    sums = _sc_encode_sums(idx, t_lin)
    pred = _tc_score_softmax(sums)
    return pred[None]
